# baseline (speedup 1.0000x reference)
import jax
import jax.numpy as jnp
from jax.experimental import pallas as pl
from jax.experimental.pallas import tpu as pltpu

BM = 1024
BN = 512


def _mm_kernel(x_ref, w_ref, b_ref, o_ref, xs_ref):
    j = pl.program_id(1)

    @pl.when(j == 0)
    def _():
        xs_ref[...] = x_ref[...].astype(jnp.bfloat16)

    wb = w_ref[...].astype(jnp.bfloat16)
    acc = jax.lax.dot_general(
        xs_ref[...], wb,
        dimension_numbers=(((1,), (1,)), ((), ())),
        preferred_element_type=jnp.float32,
    )
    o_ref[...] = acc + b_ref[...]


def kernel(x, weight, bias):
    M, K = x.shape
    N = weight.shape[0]
    b2 = bias.reshape(1, N)
    return pl.pallas_call(
        _mm_kernel,
        grid=(M // BM, N // BN),
        in_specs=[
            pl.BlockSpec((BM, K), lambda i, j: (i, 0)),
            pl.BlockSpec((BN, K), lambda i, j: (j, 0)),
            pl.BlockSpec((1, BN), lambda i, j: (0, j)),
        ],
        out_specs=pl.BlockSpec((BM, BN), lambda i, j: (i, j)),
        out_shape=jax.ShapeDtypeStruct((M, N), jnp.float32),
        scratch_shapes=[pltpu.VMEM((BM, K), jnp.bfloat16)],
        compiler_params=pltpu.CompilerParams(
            dimension_semantics=("parallel", "parallel"),
            vmem_limit_bytes=64 * 1024 * 1024,
        ),
    )(x, weight, b2)


# speedup vs baseline: 1.0148x; 1.0148x over previous
import jax
import jax.numpy as jnp
from jax.experimental import pallas as pl
from jax.experimental.pallas import tpu as pltpu

BM = 2048
BN = 512


def _mm_kernel(x_ref, w_ref, b_ref, o_ref):
    wb = w_ref[...].astype(jnp.bfloat16)
    acc = jax.lax.dot_general(
        x_ref[...], wb,
        dimension_numbers=(((1,), (1,)), ((), ())),
        preferred_element_type=jnp.float32,
    )
    o_ref[...] = acc + b_ref[...]


def kernel(x, weight, bias):
    M, K = x.shape
    N = weight.shape[0]
    xb = x.astype(jnp.bfloat16)
    b2 = bias.reshape(1, N)
    return pl.pallas_call(
        _mm_kernel,
        grid=(M // BM, N // BN),
        in_specs=[
            pl.BlockSpec((BM, K), lambda i, j: (i, 0)),
            pl.BlockSpec((BN, K), lambda i, j: (j, 0)),
            pl.BlockSpec((1, BN), lambda i, j: (0, j)),
        ],
        out_specs=pl.BlockSpec((BM, BN), lambda i, j: (i, j)),
        out_shape=jax.ShapeDtypeStruct((M, N), jnp.float32),
        compiler_params=pltpu.CompilerParams(
            dimension_semantics=("parallel", "parallel"),
            vmem_limit_bytes=63 * 1024 * 1024,
        ),
    )(xb, weight, b2)


# speedup vs baseline: 1.0473x; 1.0321x over previous
import jax
import jax.numpy as jnp
from jax.experimental import pallas as pl
from jax.experimental.pallas import tpu as pltpu

BM = 2048
BN = 512
CHUNKS = 32


def _x_chunk_copy(x_hbm, stage_ref, sem, row_base, ch, slot):
    return pltpu.make_async_copy(
        x_hbm.at[pl.ds(row_base, ch), :],
        stage_ref.at[slot],
        sem.at[slot],
    )


def _mm_kernel(x_hbm, w_ref, b_ref, o_ref, xs_ref, stage_ref, sem):
    i = pl.program_id(0)
    j = pl.program_id(1)
    ni = pl.num_programs(0)
    cur = jax.lax.rem(i, 2)
    nxt = jax.lax.rem(i + 1, 2)
    ch = BM // CHUNKS

    @pl.when((i > 0) & (j == 0))
    def _():
        c = CHUNKS - 1
        slot = c % 2
        _x_chunk_copy(x_hbm, stage_ref, sem, i * BM + c * ch, ch, slot).wait()
        xs_ref[cur, pl.ds(c * ch, ch), :] = stage_ref[slot].astype(jnp.bfloat16)

    @pl.when((i == 0) & (j == 0))
    def _():
        for c in range(CHUNKS):
            _x_chunk_copy(x_hbm, stage_ref, sem, c * ch, ch, c % 2).start()
            if c > 0:
                p = c - 1
                _x_chunk_copy(x_hbm, stage_ref, sem, p * ch, ch, p % 2).wait()
                xs_ref[0, pl.ds(p * ch, ch), :] = (
                    stage_ref[p % 2].astype(jnp.bfloat16))
        c = CHUNKS - 1
        _x_chunk_copy(x_hbm, stage_ref, sem, c * ch, ch, c % 2).wait()
        xs_ref[0, pl.ds(c * ch, ch), :] = stage_ref[c % 2].astype(jnp.bfloat16)

    @pl.when(i + 1 < ni)
    def _():
        c = j
        slot = jax.lax.rem(c, 2)

        @pl.when(c > 0)
        def _():
            p = c - 1
            pslot = jax.lax.rem(p, 2)
            _x_chunk_copy(
                x_hbm, stage_ref, sem, (i + 1) * BM + p * ch, ch, pslot).wait()
            xs_ref[nxt, pl.ds(p * ch, ch), :] = (
                stage_ref[pslot].astype(jnp.bfloat16))

        _x_chunk_copy(
            x_hbm, stage_ref, sem, (i + 1) * BM + c * ch, ch, slot).start()

    wb = w_ref[...].astype(jnp.bfloat16)
    acc = jax.lax.dot_general(
        xs_ref[cur], wb,
        dimension_numbers=(((1,), (1,)), ((), ())),
        preferred_element_type=jnp.float32,
    )
    o_ref[...] = acc + b_ref[...]


def kernel(x, weight, bias):
    M, K = x.shape
    N = weight.shape[0]
    assert N // BN == CHUNKS and BM % CHUNKS == 0
    b2 = bias.reshape(1, N)
    ch = BM // CHUNKS
    return pl.pallas_call(
        _mm_kernel,
        grid=(M // BM, N // BN),
        in_specs=[
            pl.BlockSpec(memory_space=pl.ANY),
            pl.BlockSpec((BN, K), lambda i, j: (j, 0)),
            pl.BlockSpec((1, BN), lambda i, j: (0, j)),
        ],
        out_specs=pl.BlockSpec((BM, BN), lambda i, j: (i, j)),
        out_shape=jax.ShapeDtypeStruct((M, N), jnp.float32),
        scratch_shapes=[
            pltpu.VMEM((2, BM, K), jnp.bfloat16),
            pltpu.VMEM((2, ch, K), jnp.float32),
            pltpu.SemaphoreType.DMA((2,)),
        ],
        compiler_params=pltpu.CompilerParams(
            dimension_semantics=("arbitrary", "arbitrary"),
            vmem_limit_bytes=63 * 1024 * 1024,
        ),
    )(x, weight, b2)


# speedup vs baseline: 1.0480x; 1.0007x over previous
import jax
import jax.numpy as jnp
from jax.experimental import pallas as pl
from jax.experimental.pallas import tpu as pltpu

BM = 2048
BN = 512
CHUNKS = 32


def _x_chunk_copy(x_hbm, stage_ref, sem, row_base, ch, slot):
    return pltpu.make_async_copy(
        x_hbm.at[pl.ds(row_base, ch), :],
        stage_ref.at[slot],
        sem.at[slot],
    )


def _mm_kernel(x_hbm, w_ref, b_ref, o_ref, xs_ref, stage_ref, sem):
    i = pl.program_id(0)
    j = pl.program_id(1)
    ni = pl.num_programs(0)
    cur = jax.lax.rem(i, 2)
    nxt = jax.lax.rem(i + 1, 2)
    ch = BM // CHUNKS

    @pl.when((i > 0) & (j == 0))
    def _():
        c = CHUNKS - 1
        slot = c % 2
        _x_chunk_copy(x_hbm, stage_ref, sem, i * BM + c * ch, ch, slot).wait()
        xs_ref[cur, pl.ds(c * ch, ch), :] = stage_ref[slot].astype(jnp.bfloat16)

    @pl.when((i == 0) & (j == 0))
    def _():
        for c in range(CHUNKS):
            _x_chunk_copy(x_hbm, stage_ref, sem, c * ch, ch, c % 2).start()
            if c > 0:
                p = c - 1
                _x_chunk_copy(x_hbm, stage_ref, sem, p * ch, ch, p % 2).wait()
                xs_ref[0, pl.ds(p * ch, ch), :] = (
                    stage_ref[p % 2].astype(jnp.bfloat16))
        c = CHUNKS - 1
        _x_chunk_copy(x_hbm, stage_ref, sem, c * ch, ch, c % 2).wait()
        xs_ref[0, pl.ds(c * ch, ch), :] = stage_ref[c % 2].astype(jnp.bfloat16)

    @pl.when(i + 1 < ni)
    def _():
        c = j
        slot = jax.lax.rem(c, 2)

        @pl.when(c > 0)
        def _():
            p = c - 1
            pslot = jax.lax.rem(p, 2)
            _x_chunk_copy(
                x_hbm, stage_ref, sem, (i + 1) * BM + p * ch, ch, pslot).wait()
            xs_ref[nxt, pl.ds(p * ch, ch), :] = (
                stage_ref[pslot].astype(jnp.bfloat16))

        _x_chunk_copy(
            x_hbm, stage_ref, sem, (i + 1) * BM + c * ch, ch, slot).start()

    wb = w_ref[...].astype(jnp.bfloat16)
    acc = jax.lax.dot_general(
        xs_ref[cur], wb,
        dimension_numbers=(((1,), (1,)), ((), ())),
        preferred_element_type=jnp.float32,
    )
    o_ref[...] = acc + b_ref[...]


def kernel(x, weight, bias):
    M, K = x.shape
    N = weight.shape[0]
    assert N // BN == CHUNKS and BM % CHUNKS == 0
    b2 = bias.reshape(1, N)
    ch = BM // CHUNKS
    nj1 = N // BN - 1

    def _serp(i, j):
        return jnp.where(jax.lax.rem(i, 2) == 0, j, nj1 - j)

    return pl.pallas_call(
        _mm_kernel,
        grid=(M // BM, N // BN),
        in_specs=[
            pl.BlockSpec(memory_space=pl.ANY),
            pl.BlockSpec((BN, K), lambda i, j: (_serp(i, j), 0)),
            pl.BlockSpec((1, BN), lambda i, j: (0, _serp(i, j))),
        ],
        out_specs=pl.BlockSpec((BM, BN), lambda i, j: (i, _serp(i, j))),
        out_shape=jax.ShapeDtypeStruct((M, N), jnp.float32),
        scratch_shapes=[
            pltpu.VMEM((2, BM, K), jnp.bfloat16),
            pltpu.VMEM((2, ch, K), jnp.float32),
            pltpu.SemaphoreType.DMA((2,)),
        ],
        compiler_params=pltpu.CompilerParams(
            dimension_semantics=("arbitrary", "arbitrary"),
            vmem_limit_bytes=63 * 1024 * 1024,
        ),
    )(x, weight, b2)


# speedup vs baseline: 1.1113x; 1.0604x over previous
import jax
import jax.numpy as jnp
import numpy as np
from jax.experimental import pallas as pl
from jax.experimental.pallas import tpu as pltpu
from jax.sharding import Mesh, PartitionSpec as P

try:
    from jax import shard_map as _shard_map_fn

    def _shard_map(f, mesh, in_specs, out_specs):
        return _shard_map_fn(f, mesh=mesh, in_specs=in_specs,
                             out_specs=out_specs, check_vma=False)
except ImportError:
    from jax.experimental.shard_map import shard_map as _shard_map_fn

    def _shard_map(f, mesh, in_specs, out_specs):
        return _shard_map_fn(f, mesh=mesh, in_specs=in_specs,
                             out_specs=out_specs, check_rep=False)

BM = 2048
BN = 512
CHUNKS = 32


def _x_chunk_copy(x_hbm, stage_ref, sem, row_base, ch, slot):
    return pltpu.make_async_copy(
        x_hbm.at[pl.ds(row_base, ch), :],
        stage_ref.at[slot],
        sem.at[slot],
    )


def _mm_kernel(x_hbm, w_ref, b_ref, o_ref, xs_ref, stage_ref, sem):
    i = pl.program_id(0)
    j = pl.program_id(1)
    ni = pl.num_programs(0)
    cur = jax.lax.rem(i, 2)
    nxt = jax.lax.rem(i + 1, 2)
    ch = BM // CHUNKS

    @pl.when((i > 0) & (j == 0))
    def _():
        c = CHUNKS - 1
        slot = c % 2
        _x_chunk_copy(x_hbm, stage_ref, sem, i * BM + c * ch, ch, slot).wait()
        xs_ref[cur, pl.ds(c * ch, ch), :] = stage_ref[slot].astype(jnp.bfloat16)

    @pl.when((i == 0) & (j == 0))
    def _():
        for c in range(CHUNKS):
            _x_chunk_copy(x_hbm, stage_ref, sem, c * ch, ch, c % 2).start()
            if c > 0:
                p = c - 1
                _x_chunk_copy(x_hbm, stage_ref, sem, p * ch, ch, p % 2).wait()
                xs_ref[0, pl.ds(p * ch, ch), :] = (
                    stage_ref[p % 2].astype(jnp.bfloat16))
        c = CHUNKS - 1
        _x_chunk_copy(x_hbm, stage_ref, sem, c * ch, ch, c % 2).wait()
        xs_ref[0, pl.ds(c * ch, ch), :] = stage_ref[c % 2].astype(jnp.bfloat16)

    @pl.when(i + 1 < ni)
    def _():
        c = j
        slot = jax.lax.rem(c, 2)

        @pl.when(c > 0)
        def _():
            p = c - 1
            pslot = jax.lax.rem(p, 2)
            _x_chunk_copy(
                x_hbm, stage_ref, sem, (i + 1) * BM + p * ch, ch, pslot).wait()
            xs_ref[nxt, pl.ds(p * ch, ch), :] = (
                stage_ref[pslot].astype(jnp.bfloat16))

        _x_chunk_copy(
            x_hbm, stage_ref, sem, (i + 1) * BM + c * ch, ch, slot).start()

    wb = w_ref[...].astype(jnp.bfloat16)
    acc = jax.lax.dot_general(
        xs_ref[cur], wb,
        dimension_numbers=(((1,), (1,)), ((), ())),
        preferred_element_type=jnp.float32,
    )
    o_ref[...] = acc + b_ref[...]


def _matmul_call(x, weight, bias):
    M, K = x.shape
    N = weight.shape[0]
    assert N // BN == CHUNKS and BM % CHUNKS == 0
    b2 = bias.reshape(1, N)
    ch = BM // CHUNKS
    nj1 = N // BN - 1

    def _serp(i, j):
        return jnp.where(jax.lax.rem(i, 2) == 0, j, nj1 - j)

    return pl.pallas_call(
        _mm_kernel,
        grid=(M // BM, N // BN),
        in_specs=[
            pl.BlockSpec(memory_space=pl.ANY),
            pl.BlockSpec((BN, K), lambda i, j: (_serp(i, j), 0)),
            pl.BlockSpec((1, BN), lambda i, j: (0, _serp(i, j))),
        ],
        out_specs=pl.BlockSpec((BM, BN), lambda i, j: (i, _serp(i, j))),
        out_shape=jax.ShapeDtypeStruct((M, N), jnp.float32),
        scratch_shapes=[
            pltpu.VMEM((2, BM, K), jnp.bfloat16),
            pltpu.VMEM((2, ch, K), jnp.float32),
            pltpu.SemaphoreType.DMA((2,)),
        ],
        compiler_params=pltpu.CompilerParams(
            dimension_semantics=("arbitrary", "arbitrary"),
            vmem_limit_bytes=63 * 1024 * 1024,
        ),
    )(x, weight, b2)


def kernel(x, weight, bias):
    devs = jax.devices()
    if len(devs) < 2 or x.shape[0] % (2 * BM) != 0:
        return _matmul_call(x, weight, bias)
    mesh = Mesh(np.array(devs[:2]), ("d",))
    f = _shard_map(
        _matmul_call,
        mesh,
        in_specs=(P("d", None), P(None, None), P(None)),
        out_specs=P("d", None),
    )
    return f(x, weight, bias)
